# Initial kernel scaffold; baseline (speedup 1.0000x reference)
import functools

import jax
import jax.numpy as jnp
from jax import lax
from jax.experimental import pallas as pl
from jax.experimental.pallas import tpu as pltpu
from jax.experimental.pallas import tpu_sc as plsc

N = 50000
E = 800000
B = 512
T = 10
H = 64
O = 15

NC = 2
NS = 16
NW = NC * NS

REAL_HALF = 25000
HALF = 25088
NPAD = 2 * HALF
PAD = HALF - REAL_HALF
TRASH = 25040
RPT = HALF // NS

EPT = 25600
EPAD = NW * EPT
CHUNK = 128

GACC = 640
GTRASH = 520
GRPT = GACC // NS
PCHUNK = 112
NPT = NPAD // NW

ROW_BLK = 512
GRID = NPAD // ROW_BLK


def _sc_mesh():
    return plsc.VectorSubcoreMesh(
        core_axis_name="c", subcore_axis_name="s", num_cores=NC, num_subcores=NS
    )


@functools.lru_cache(maxsize=None)
def _build_agg():
    @functools.partial(
        pl.kernel,
        out_type=jax.ShapeDtypeStruct((NPAD, H), jnp.float32),
        mesh=_sc_mesh(),
        scratch_types=[
            pltpu.VMEM((CHUNK,), jnp.int32),
            pltpu.VMEM((CHUNK,), jnp.int32),
            pltpu.VMEM((CHUNK,), jnp.int32),
            pltpu.VMEM((CHUNK, H), jnp.float32),
            pltpu.VMEM_SHARED((HALF, H), jnp.float32),
            pltpu.SemaphoreType.DMA,
        ],
    )
    def agg(g_hbm, src_hbm, dst_hbm, ztile_hbm, out_hbm,
            src_v, dst_v, ld_v, rows_v, acc, sem):
        c = lax.axis_index("c")
        s = lax.axis_index("s")
        wid = s * NC + c
        pltpu.sync_copy(ztile_hbm, acc.at[pl.ds(s * RPT, RPT)])
        plsc.subcore_barrier()
        core_base = c * HALF
        ebase = wid * EPT

        def eiter(i, carry):
            off = ebase + i * CHUNK
            pltpu.sync_copy(src_hbm.at[pl.ds(off, CHUNK)], src_v)
            pltpu.sync_copy(dst_hbm.at[pl.ds(off, CHUNK)], dst_v)
            for j in range(CHUNK // 16):
                d = dst_v[pl.ds(j * 16, 16)]
                ld = d - core_base
                ok = (ld >= 0) & (ld < REAL_HALF)
                ld_v[pl.ds(j * 16, 16)] = jnp.where(ok, ld, TRASH)
            pltpu.async_copy(g_hbm.at[src_v], rows_v, sem).wait()
            pltpu.sync_copy(rows_v, acc.at[ld_v], add=True)
            return carry

        lax.fori_loop(0, EPT // CHUNK, eiter, 0)
        plsc.subcore_barrier()
        pltpu.sync_copy(
            acc.at[pl.ds(s * RPT, RPT)],
            out_hbm.at[pl.ds(core_base + s * RPT, RPT)],
        )

    return agg


@functools.lru_cache(maxsize=None)
def _build_deg():
    @functools.partial(
        pl.kernel,
        out_type=jax.ShapeDtypeStruct((NPAD, H), jnp.float32),
        mesh=_sc_mesh(),
        scratch_types=[
            pltpu.VMEM((CHUNK,), jnp.int32),
            pltpu.VMEM((CHUNK,), jnp.int32),
            pltpu.VMEM((CHUNK, H), jnp.float32),
            pltpu.VMEM_SHARED((HALF, H), jnp.float32),
        ],
    )
    def deg(dst_hbm, ztile_hbm, ones_hbm, out_hbm, dst_v, ld_v, ones_v, acc):
        c = lax.axis_index("c")
        s = lax.axis_index("s")
        wid = s * NC + c
        pltpu.sync_copy(ztile_hbm, acc.at[pl.ds(s * RPT, RPT)])
        pltpu.sync_copy(ones_hbm, ones_v)
        plsc.subcore_barrier()
        core_base = c * HALF
        ebase = wid * EPT

        def eiter(i, carry):
            off = ebase + i * CHUNK
            pltpu.sync_copy(dst_hbm.at[pl.ds(off, CHUNK)], dst_v)
            for j in range(CHUNK // 16):
                d = dst_v[pl.ds(j * 16, 16)]
                ld = d - core_base
                ok = (ld >= 0) & (ld < REAL_HALF)
                ld_v[pl.ds(j * 16, 16)] = jnp.where(ok, ld, TRASH)
            pltpu.sync_copy(ones_v, acc.at[ld_v], add=True)
            return carry

        lax.fori_loop(0, EPT // CHUNK, eiter, 0)
        plsc.subcore_barrier()
        pltpu.sync_copy(
            acc.at[pl.ds(s * RPT, RPT)],
            out_hbm.at[pl.ds(core_base + s * RPT, RPT)],
        )

    return deg


@functools.lru_cache(maxsize=None)
def _build_pool():
    @functools.partial(
        pl.kernel,
        out_type=(
            jax.ShapeDtypeStruct((NC, GACC, H), jnp.float32),
            jax.ShapeDtypeStruct((NC, GACC, H), jnp.float32),
        ),
        mesh=_sc_mesh(),
        scratch_types=[
            pltpu.VMEM((PCHUNK,), jnp.int32),
            pltpu.VMEM((PCHUNK, H), jnp.float32),
            pltpu.VMEM((PCHUNK, H), jnp.float32),
            pltpu.VMEM_SHARED((GACC, H), jnp.float32),
            pltpu.VMEM_SHARED((GACC, H), jnp.float32),
        ],
    )
    def pool(h_hbm, b_hbm, zg_hbm, onesp_hbm, outp_hbm, outc_hbm,
             b_v, rows_v, ones_v, accp, accc):
        c = lax.axis_index("c")
        s = lax.axis_index("s")
        wid = s * NC + c
        pltpu.sync_copy(zg_hbm, accp.at[pl.ds(s * GRPT, GRPT)])
        pltpu.sync_copy(zg_hbm, accc.at[pl.ds(s * GRPT, GRPT)])
        pltpu.sync_copy(onesp_hbm, ones_v)
        plsc.subcore_barrier()
        nbase = wid * NPT

        def piter(i, carry):
            off = nbase + i * PCHUNK
            pltpu.sync_copy(b_hbm.at[pl.ds(off, PCHUNK)], b_v)
            pltpu.sync_copy(h_hbm.at[pl.ds(off, PCHUNK)], rows_v)
            pltpu.sync_copy(rows_v, accp.at[b_v], add=True)
            pltpu.sync_copy(ones_v, accc.at[b_v], add=True)
            return carry

        lax.fori_loop(0, NPT // PCHUNK, piter, 0)
        plsc.subcore_barrier()
        pltpu.sync_copy(accp.at[pl.ds(s * GRPT, GRPT)],
                        outp_hbm.at[c, pl.ds(s * GRPT, GRPT)])
        pltpu.sync_copy(accc.at[pl.ds(s * GRPT, GRPT)],
                        outc_hbm.at[c, pl.ds(s * GRPT, GRPT)])

    return pool


def _l1_body(x_ref, deg_ref, embp_ref, w1_ref, g_ref, dinv_ref):
    pid = pl.program_id(0)
    x = x_ref[...]
    tt = lax.broadcasted_iota(jnp.float32, (1, 16), 1)
    oh = (x == tt).astype(jnp.float32)
    table = jnp.dot(embp_ref[...], w1_ref[...],
                    preferred_element_type=jnp.float32)
    cnt = deg_ref[:, 0:1]
    rid = pid * ROW_BLK + lax.broadcasted_iota(jnp.int32, (ROW_BLK, 1), 0)
    valid = (rid % HALF) < REAL_HALF
    dinv = jnp.where(valid, lax.rsqrt(1.0 + cnt), 0.0)
    g_ref[...] = jnp.dot(oh, table, preferred_element_type=jnp.float32) * dinv
    dinv_ref[...] = dinv


@functools.lru_cache(maxsize=None)
def _build_l1():
    return pl.pallas_call(
        _l1_body,
        grid=(GRID,),
        in_specs=[
            pl.BlockSpec((ROW_BLK, 1), lambda i: (i, 0)),
            pl.BlockSpec((ROW_BLK, H), lambda i: (i, 0)),
            pl.BlockSpec((16, H), lambda i: (0, 0)),
            pl.BlockSpec((H, H), lambda i: (0, 0)),
        ],
        out_specs=[
            pl.BlockSpec((ROW_BLK, H), lambda i: (i, 0)),
            pl.BlockSpec((ROW_BLK, 1), lambda i: (i, 0)),
        ],
        out_shape=[
            jax.ShapeDtypeStruct((NPAD, H), jnp.float32),
            jax.ShapeDtypeStruct((NPAD, 1), jnp.float32),
        ],
    )


def _l23_body(s_ref, g_ref, dinv_ref, b_ref, w_ref, gout_ref):
    dinv = dinv_ref[...]
    h = jnp.maximum(dinv * (s_ref[...] + g_ref[...]) + b_ref[...], 0.0)
    gout_ref[...] = jnp.dot(h, w_ref[...],
                            preferred_element_type=jnp.float32) * dinv


@functools.lru_cache(maxsize=None)
def _build_l23():
    return pl.pallas_call(
        _l23_body,
        grid=(GRID,),
        in_specs=[
            pl.BlockSpec((ROW_BLK, H), lambda i: (i, 0)),
            pl.BlockSpec((ROW_BLK, H), lambda i: (i, 0)),
            pl.BlockSpec((ROW_BLK, 1), lambda i: (i, 0)),
            pl.BlockSpec((1, H), lambda i: (0, 0)),
            pl.BlockSpec((H, H), lambda i: (0, 0)),
        ],
        out_specs=pl.BlockSpec((ROW_BLK, H), lambda i: (i, 0)),
        out_shape=jax.ShapeDtypeStruct((NPAD, H), jnp.float32),
    )


def _comb_body(s_ref, g_ref, dinv_ref, b_ref, h_ref):
    h_ref[...] = jnp.maximum(
        dinv_ref[...] * (s_ref[...] + g_ref[...]) + b_ref[...], 0.0)


@functools.lru_cache(maxsize=None)
def _build_comb():
    return pl.pallas_call(
        _comb_body,
        grid=(GRID,),
        in_specs=[
            pl.BlockSpec((ROW_BLK, H), lambda i: (i, 0)),
            pl.BlockSpec((ROW_BLK, H), lambda i: (i, 0)),
            pl.BlockSpec((ROW_BLK, 1), lambda i: (i, 0)),
            pl.BlockSpec((1, H), lambda i: (0, 0)),
        ],
        out_specs=pl.BlockSpec((ROW_BLK, H), lambda i: (i, 0)),
        out_shape=jax.ShapeDtypeStruct((NPAD, H), jnp.float32),
    )


def _head_body(p_ref, cnt_ref, conc_ref, wc_ref, bc_ref,
               wf1_ref, bf1_ref, wf2_ref, bf2_ref, o_ref):
    sums = p_ref[0] + p_ref[1]
    cnts = cnt_ref[0, :, 0:1] + cnt_ref[1, :, 0:1]
    ge = sums[:B] / jnp.maximum(cnts[:B], 1.0)
    conc_e = conc_ref[...] * wc_ref[...] + bc_ref[...]
    h2 = jnp.maximum(
        jnp.dot(ge, wf1_ref[:H], preferred_element_type=jnp.float32)
        + jnp.dot(conc_e, wf1_ref[H:], preferred_element_type=jnp.float32)
        + bf1_ref[...], 0.0)
    o_ref[...] = jnp.dot(h2, wf2_ref[...],
                         preferred_element_type=jnp.float32) + bf2_ref[...]


@functools.lru_cache(maxsize=None)
def _build_head():
    return pl.pallas_call(
        _head_body,
        out_shape=jax.ShapeDtypeStruct((B, 128), jnp.float32),
    )


def kernel(x, edge_index, batch, concentration, emb,
           W1, b1, W2, b2, W3, b3, Wc, bc, Wf1, bf1, Wf2, bf2):
    f32 = jnp.float32
    src = edge_index[0]
    dst = edge_index[1]
    srcp = src + PAD * (src >= REAL_HALF).astype(jnp.int32)
    dstp = dst + PAD * (dst >= REAL_HALF).astype(jnp.int32)
    srcp = jnp.concatenate([srcp, jnp.zeros((EPAD - E,), jnp.int32)])
    dstp = jnp.concatenate([dstp, jnp.full((EPAD - E,), -1, jnp.int32)])

    padi = jnp.zeros((PAD,), jnp.int32)
    xp = jnp.concatenate([x[:REAL_HALF], padi, x[REAL_HALF:], padi])
    xp = xp.astype(f32).reshape(NPAD, 1)
    padb = jnp.full((PAD,), GTRASH, jnp.int32)
    batchp = jnp.concatenate([batch[:REAL_HALF], padb, batch[REAL_HALF:], padb])

    ztile = jnp.zeros((RPT, H), f32)
    zg = jnp.zeros((GRPT, H), f32)
    ones_chunk = jnp.ones((CHUNK, H), f32)
    ones_p = jnp.ones((PCHUNK, H), f32)
    embp = jnp.pad(emb, ((0, 16 - T), (0, 0)))
    wf2p = jnp.pad(Wf2, ((0, 0), (0, 128 - O)))
    bf2p = jnp.pad(bf2, (0, 128 - O)).reshape(1, 128)

    degr = _build_deg()(dstp, ztile, ones_chunk)
    g1, dinv = _build_l1()(xp, degr, embp, W1)
    s1 = _build_agg()(g1, srcp, dstp, ztile)
    g2 = _build_l23()(s1, g1, dinv, b1.reshape(1, H), W2)
    s2 = _build_agg()(g2, srcp, dstp, ztile)
    g3 = _build_l23()(s2, g2, dinv, b2.reshape(1, H), W3)
    s3 = _build_agg()(g3, srcp, dstp, ztile)
    h3 = _build_comb()(s3, g3, dinv, b3.reshape(1, H))
    p, cnt = _build_pool()(h3, batchp, zg, ones_p)
    outp = _build_head()(p, cnt, concentration.reshape(B, 1), Wc,
                         bc.reshape(1, H), Wf1, bf1.reshape(1, H), wf2p, bf2p)
    return outp[:, :O]


# speedup vs baseline: 5.7467x; 5.7467x over previous
import functools

import jax
import jax.numpy as jnp
from jax import lax
from jax.experimental import pallas as pl
from jax.experimental.pallas import tpu as pltpu
from jax.experimental.pallas import tpu_sc as plsc

N = 50000
E = 800000
B = 512
T = 10
H = 64
O = 15

NC = 2
NS = 16
NW = NC * NS

REAL_HALF = 25000
HALF = 25088
NPAD = 2 * HALF
PAD = HALF - REAL_HALF
TRASH = 25040
RPT = HALF // NS

EPAD = 819200
EPT = EPAD // NS
CHUNK = 128

GACC = 640
GTRASH = 520
GRPT = GACC // NS
PCHUNK = 112
NPT = NPAD // NW

ROW_BLK = 512
GRID = NPAD // ROW_BLK


def _sc_mesh():
    return plsc.VectorSubcoreMesh(
        core_axis_name="c", subcore_axis_name="s", num_cores=NC, num_subcores=NS
    )


@functools.lru_cache(maxsize=None)
def _build_agg():
    @functools.partial(
        pl.kernel,
        out_type=jax.ShapeDtypeStruct((NPAD, H), jnp.float32),
        mesh=_sc_mesh(),
        compiler_params=pltpu.CompilerParams(use_tc_tiling_on_sc=False),
        scratch_types=[
            pltpu.VMEM((CHUNK,), jnp.int32),
            pltpu.VMEM((CHUNK,), jnp.int32),
            pltpu.VMEM((CHUNK, H), jnp.float32),
            pltpu.VMEM_SHARED((HALF, H), jnp.float32),
            pltpu.SemaphoreType.DMA,
        ],
    )
    def agg(g_hbm, src_hbm, dstl_hbm, ztile_hbm, out_hbm,
            src_v, ld_v, rows_v, acc, sem):
        c = lax.axis_index("c")
        s = lax.axis_index("s")
        pltpu.sync_copy(ztile_hbm, acc.at[pl.ds(s * RPT, RPT)])
        plsc.subcore_barrier()
        core_base = c * HALF
        ebase = s * EPT

        def eiter(i, carry):
            off = ebase + i * CHUNK
            pltpu.sync_copy(src_hbm.at[pl.ds(off, CHUNK)], src_v)
            pltpu.sync_copy(dstl_hbm.at[c, pl.ds(off, CHUNK)], ld_v)
            pltpu.async_copy(g_hbm.at[src_v], rows_v, sem).wait()
            pltpu.sync_copy(rows_v, acc.at[ld_v], add=True)
            return carry

        lax.fori_loop(0, EPT // CHUNK, eiter, 0)
        plsc.subcore_barrier()
        pltpu.sync_copy(
            acc.at[pl.ds(s * RPT, RPT)],
            out_hbm.at[pl.ds(core_base + s * RPT, RPT)],
        )

    return agg


@functools.lru_cache(maxsize=None)
def _build_deg():
    @functools.partial(
        pl.kernel,
        out_type=jax.ShapeDtypeStruct((NPAD, H), jnp.float32),
        mesh=_sc_mesh(),
        compiler_params=pltpu.CompilerParams(use_tc_tiling_on_sc=False),
        scratch_types=[
            pltpu.VMEM((CHUNK,), jnp.int32),
            pltpu.VMEM((CHUNK, H), jnp.float32),
            pltpu.VMEM_SHARED((HALF, H), jnp.float32),
        ],
    )
    def deg(dstl_hbm, ztile_hbm, ones_hbm, out_hbm, ld_v, ones_v, acc):
        c = lax.axis_index("c")
        s = lax.axis_index("s")
        pltpu.sync_copy(ztile_hbm, acc.at[pl.ds(s * RPT, RPT)])
        pltpu.sync_copy(ones_hbm, ones_v)
        plsc.subcore_barrier()
        core_base = c * HALF
        ebase = s * EPT

        def eiter(i, carry):
            off = ebase + i * CHUNK
            pltpu.sync_copy(dstl_hbm.at[c, pl.ds(off, CHUNK)], ld_v)
            pltpu.sync_copy(ones_v, acc.at[ld_v], add=True)
            return carry

        lax.fori_loop(0, EPT // CHUNK, eiter, 0)
        plsc.subcore_barrier()
        pltpu.sync_copy(
            acc.at[pl.ds(s * RPT, RPT)],
            out_hbm.at[pl.ds(core_base + s * RPT, RPT)],
        )

    return deg


@functools.lru_cache(maxsize=None)
def _build_pool():
    @functools.partial(
        pl.kernel,
        out_type=(
            jax.ShapeDtypeStruct((NC, GACC, H), jnp.float32),
            jax.ShapeDtypeStruct((NC, GACC, H), jnp.float32),
        ),
        mesh=_sc_mesh(),
        compiler_params=pltpu.CompilerParams(use_tc_tiling_on_sc=False),
        scratch_types=[
            pltpu.VMEM((PCHUNK,), jnp.int32),
            pltpu.VMEM((PCHUNK, H), jnp.float32),
            pltpu.VMEM((PCHUNK, H), jnp.float32),
            pltpu.VMEM_SHARED((GACC, H), jnp.float32),
            pltpu.VMEM_SHARED((GACC, H), jnp.float32),
        ],
    )
    def pool(h_hbm, b_hbm, zg_hbm, onesp_hbm, outp_hbm, outc_hbm,
             b_v, rows_v, ones_v, accp, accc):
        c = lax.axis_index("c")
        s = lax.axis_index("s")
        wid = s * NC + c
        pltpu.sync_copy(zg_hbm, accp.at[pl.ds(s * GRPT, GRPT)])
        pltpu.sync_copy(zg_hbm, accc.at[pl.ds(s * GRPT, GRPT)])
        pltpu.sync_copy(onesp_hbm, ones_v)
        plsc.subcore_barrier()
        nbase = wid * NPT

        def piter(i, carry):
            off = nbase + i * PCHUNK
            pltpu.sync_copy(b_hbm.at[pl.ds(off, PCHUNK)], b_v)
            pltpu.sync_copy(h_hbm.at[pl.ds(off, PCHUNK)], rows_v)
            pltpu.sync_copy(rows_v, accp.at[b_v], add=True)
            pltpu.sync_copy(ones_v, accc.at[b_v], add=True)
            return carry

        lax.fori_loop(0, NPT // PCHUNK, piter, 0)
        plsc.subcore_barrier()
        pltpu.sync_copy(accp.at[pl.ds(s * GRPT, GRPT)],
                        outp_hbm.at[c, pl.ds(s * GRPT, GRPT)])
        pltpu.sync_copy(accc.at[pl.ds(s * GRPT, GRPT)],
                        outc_hbm.at[c, pl.ds(s * GRPT, GRPT)])

    return pool


def _l1_body(x_ref, deg_ref, embp_ref, w1_ref, g_ref, dinv_ref):
    pid = pl.program_id(0)
    x = x_ref[...]
    tt = lax.broadcasted_iota(jnp.int32, (1, 16), 1).astype(jnp.float32)
    oh = (x == tt).astype(jnp.float32)
    table = jnp.dot(embp_ref[...], w1_ref[...],
                    preferred_element_type=jnp.float32,
                    precision=lax.Precision.HIGHEST)
    cnt = deg_ref[:, 0:1]
    rid = pid * ROW_BLK + lax.broadcasted_iota(jnp.int32, (ROW_BLK, 1), 0)
    valid = (rid % HALF) < REAL_HALF
    dinv = jnp.where(valid, lax.rsqrt(1.0 + cnt), 0.0)
    g_ref[...] = jnp.dot(oh, table, preferred_element_type=jnp.float32,
                    precision=lax.Precision.HIGHEST) * dinv
    dinv_ref[...] = dinv


@functools.lru_cache(maxsize=None)
def _build_l1():
    return pl.pallas_call(
        _l1_body,
        grid=(GRID,),
        in_specs=[
            pl.BlockSpec((ROW_BLK, 1), lambda i: (i, 0)),
            pl.BlockSpec((ROW_BLK, H), lambda i: (i, 0)),
            pl.BlockSpec((16, H), lambda i: (0, 0)),
            pl.BlockSpec((H, H), lambda i: (0, 0)),
        ],
        out_specs=[
            pl.BlockSpec((ROW_BLK, H), lambda i: (i, 0)),
            pl.BlockSpec((ROW_BLK, 1), lambda i: (i, 0)),
        ],
        out_shape=[
            jax.ShapeDtypeStruct((NPAD, H), jnp.float32),
            jax.ShapeDtypeStruct((NPAD, 1), jnp.float32),
        ],
    )


def _l23_body(s_ref, g_ref, dinv_ref, b_ref, w_ref, gout_ref):
    dinv = dinv_ref[...]
    h = jnp.maximum(dinv * (s_ref[...] + g_ref[...]) + b_ref[...], 0.0)
    gout_ref[...] = jnp.dot(h, w_ref[...],
                            preferred_element_type=jnp.float32,
                    precision=lax.Precision.HIGHEST) * dinv


@functools.lru_cache(maxsize=None)
def _build_l23():
    return pl.pallas_call(
        _l23_body,
        grid=(GRID,),
        in_specs=[
            pl.BlockSpec((ROW_BLK, H), lambda i: (i, 0)),
            pl.BlockSpec((ROW_BLK, H), lambda i: (i, 0)),
            pl.BlockSpec((ROW_BLK, 1), lambda i: (i, 0)),
            pl.BlockSpec((1, H), lambda i: (0, 0)),
            pl.BlockSpec((H, H), lambda i: (0, 0)),
        ],
        out_specs=pl.BlockSpec((ROW_BLK, H), lambda i: (i, 0)),
        out_shape=jax.ShapeDtypeStruct((NPAD, H), jnp.float32),
    )


def _comb_body(s_ref, g_ref, dinv_ref, b_ref, h_ref):
    h_ref[...] = jnp.maximum(
        dinv_ref[...] * (s_ref[...] + g_ref[...]) + b_ref[...], 0.0)


@functools.lru_cache(maxsize=None)
def _build_comb():
    return pl.pallas_call(
        _comb_body,
        grid=(GRID,),
        in_specs=[
            pl.BlockSpec((ROW_BLK, H), lambda i: (i, 0)),
            pl.BlockSpec((ROW_BLK, H), lambda i: (i, 0)),
            pl.BlockSpec((ROW_BLK, 1), lambda i: (i, 0)),
            pl.BlockSpec((1, H), lambda i: (0, 0)),
        ],
        out_specs=pl.BlockSpec((ROW_BLK, H), lambda i: (i, 0)),
        out_shape=jax.ShapeDtypeStruct((NPAD, H), jnp.float32),
    )


def _head_body(p_ref, cnt_ref, conc_ref, wc_ref, bc_ref,
               wf1_ref, bf1_ref, wf2_ref, bf2_ref, o_ref):
    sums = p_ref[0] + p_ref[1]
    cnts = cnt_ref[0, :, 0:1] + cnt_ref[1, :, 0:1]
    ge = sums[:B] / jnp.maximum(cnts[:B], 1.0)
    conc_e = conc_ref[...] * wc_ref[...] + bc_ref[...]
    h2 = jnp.maximum(
        jnp.dot(ge, wf1_ref[:H], preferred_element_type=jnp.float32,
                    precision=lax.Precision.HIGHEST)
        + jnp.dot(conc_e, wf1_ref[H:], preferred_element_type=jnp.float32,
                    precision=lax.Precision.HIGHEST)
        + bf1_ref[...], 0.0)
    o_ref[...] = jnp.dot(h2, wf2_ref[...],
                         preferred_element_type=jnp.float32,
                    precision=lax.Precision.HIGHEST) + bf2_ref[...]


@functools.lru_cache(maxsize=None)
def _build_head():
    return pl.pallas_call(
        _head_body,
        out_shape=jax.ShapeDtypeStruct((B, 128), jnp.float32),
    )


def kernel(x, edge_index, batch, concentration, emb,
           W1, b1, W2, b2, W3, b3, Wc, bc, Wf1, bf1, Wf2, bf2):
    f32 = jnp.float32
    src = edge_index[0]
    dst = edge_index[1]
    srcp = src + PAD * (src >= REAL_HALF).astype(jnp.int32)
    dstp = dst + PAD * (dst >= REAL_HALF).astype(jnp.int32)
    srcp = jnp.concatenate([srcp, jnp.zeros((EPAD - E,), jnp.int32)])
    dstp = jnp.concatenate([dstp, jnp.full((EPAD - E,), -1, jnp.int32)])
    ld0 = jnp.where((dstp >= 0) & (dstp < REAL_HALF), dstp, TRASH)
    ld1m = dstp - HALF
    ld1 = jnp.where((ld1m >= 0) & (ld1m < REAL_HALF), ld1m, TRASH)
    dstl = jnp.stack([ld0, ld1])

    padi = jnp.zeros((PAD,), jnp.int32)
    xp = jnp.concatenate([x[:REAL_HALF], padi, x[REAL_HALF:], padi])
    xp = xp.astype(f32).reshape(NPAD, 1)
    padb = jnp.full((PAD,), GTRASH, jnp.int32)
    batchp = jnp.concatenate([batch[:REAL_HALF], padb, batch[REAL_HALF:], padb])

    ztile = jnp.zeros((RPT, H), f32)
    zg = jnp.zeros((GRPT, H), f32)
    ones_chunk = jnp.ones((CHUNK, H), f32)
    ones_p = jnp.ones((PCHUNK, H), f32)
    embp = jnp.pad(emb, ((0, 16 - T), (0, 0)))
    wf2p = jnp.pad(Wf2, ((0, 0), (0, 128 - O)))
    bf2p = jnp.pad(bf2, (0, 128 - O)).reshape(1, 128)

    degr = _build_deg()(dstl, ztile, ones_chunk)
    g1, dinv = _build_l1()(xp, degr, embp, W1)
    s1 = _build_agg()(g1, srcp, dstl, ztile)
    g2 = _build_l23()(s1, g1, dinv, b1.reshape(1, H), W2)
    s2 = _build_agg()(g2, srcp, dstl, ztile)
    g3 = _build_l23()(s2, g2, dinv, b2.reshape(1, H), W3)
    s3 = _build_agg()(g3, srcp, dstl, ztile)
    h3 = _build_comb()(s3, g3, dinv, b3.reshape(1, H))
    p, cnt = _build_pool()(h3, batchp, zg, ones_p)
    outp = _build_head()(p, cnt, concentration.reshape(B, 1), Wc,
                         bc.reshape(1, H), Wf1, bf1.reshape(1, H), wf2p, bf2p)
    return outp[:, :O]


# speedup vs baseline: 8.2869x; 1.4420x over previous
import functools

import jax
import jax.numpy as jnp
from jax import lax
from jax.experimental import pallas as pl
from jax.experimental.pallas import tpu as pltpu
from jax.experimental.pallas import tpu_sc as plsc

N = 50000
E = 800000
B = 512
T = 10
H = 64
O = 15

NC = 2
NS = 16
NW = NC * NS

REAL_HALF = 25000
HALF = 25088
NPAD = 2 * HALF
PAD = HALF - REAL_HALF
TRASH = 25040
RPT = NPAD // NS

EPAD = 819200
EPT = EPAD // NS
EPC = EPAD // NC
EPW = EPC // NS
CHUNK = 128
DCOL = 16

GACC = 640
GTRASH = 520
GRPT = GACC // NS
PCHUNK = 112
NPT = NPAD // NW

ROW_BLK = 512
GRID = NPAD // ROW_BLK


def _sc_mesh():
    return plsc.VectorSubcoreMesh(
        core_axis_name="c", subcore_axis_name="s", num_cores=NC, num_subcores=NS
    )


@functools.lru_cache(maxsize=None)
def _build_agg():
    @functools.partial(
        pl.kernel,
        out_type=jax.ShapeDtypeStruct((NC, NPAD, 32), jnp.float32),
        mesh=_sc_mesh(),
        compiler_params=pltpu.CompilerParams(use_tc_tiling_on_sc=False),
        scratch_types=[
            pltpu.VMEM((CHUNK,), jnp.int32),
            pltpu.VMEM((CHUNK,), jnp.int32),
            pltpu.VMEM((CHUNK, 32), jnp.float32),
            pltpu.VMEM_SHARED((NPAD, 32), jnp.float32),
            pltpu.SemaphoreType.DMA,
        ],
    )
    def agg(g_hbm, src_hbm, dst_hbm, ztile_hbm, out_hbm,
            src_v, d_v, rows_v, acc, sem):
        c = lax.axis_index("c")
        s = lax.axis_index("s")
        pltpu.sync_copy(ztile_hbm, acc.at[pl.ds(s * RPT, RPT)])
        plsc.subcore_barrier()
        ebase = s * EPT

        def eiter(i, carry):
            off = ebase + i * CHUNK
            pltpu.sync_copy(src_hbm.at[pl.ds(off, CHUNK)], src_v)
            pltpu.sync_copy(dst_hbm.at[pl.ds(off, CHUNK)], d_v)
            pltpu.async_copy(g_hbm.at[c].at[src_v], rows_v, sem).wait()
            pltpu.sync_copy(rows_v, acc.at[d_v], add=True)
            return carry

        lax.fori_loop(0, EPT // CHUNK, eiter, 0)
        plsc.subcore_barrier()
        pltpu.sync_copy(
            acc.at[pl.ds(s * RPT, RPT)],
            out_hbm.at[c, pl.ds(s * RPT, RPT)],
        )

    return agg


@functools.lru_cache(maxsize=None)
def _build_deg():
    @functools.partial(
        pl.kernel,
        out_type=jax.ShapeDtypeStruct((NC, NPAD, DCOL), jnp.float32),
        mesh=_sc_mesh(),
        compiler_params=pltpu.CompilerParams(use_tc_tiling_on_sc=False),
        scratch_types=[
            pltpu.VMEM((CHUNK,), jnp.int32),
            pltpu.VMEM((CHUNK, DCOL), jnp.float32),
            pltpu.VMEM_SHARED((NPAD, DCOL), jnp.float32),
        ],
    )
    def deg(dst_hbm, ztile_hbm, ones_hbm, out_hbm, d_v, ones_v, acc):
        c = lax.axis_index("c")
        s = lax.axis_index("s")
        pltpu.sync_copy(ztile_hbm, acc.at[pl.ds(s * RPT, RPT)])
        pltpu.sync_copy(ones_hbm, ones_v)
        plsc.subcore_barrier()
        ebase = c * EPC + s * EPW

        def eiter(i, carry):
            off = ebase + i * CHUNK
            pltpu.sync_copy(dst_hbm.at[pl.ds(off, CHUNK)], d_v)
            pltpu.sync_copy(ones_v, acc.at[d_v], add=True)
            return carry

        lax.fori_loop(0, EPW // CHUNK, eiter, 0)
        plsc.subcore_barrier()
        pltpu.sync_copy(
            acc.at[pl.ds(s * RPT, RPT)],
            out_hbm.at[c, pl.ds(s * RPT, RPT)],
        )

    return deg


@functools.lru_cache(maxsize=None)
def _build_pool():
    @functools.partial(
        pl.kernel,
        out_type=(
            jax.ShapeDtypeStruct((NC, GACC, H), jnp.float32),
            jax.ShapeDtypeStruct((NC, GACC, H), jnp.float32),
        ),
        mesh=_sc_mesh(),
        compiler_params=pltpu.CompilerParams(use_tc_tiling_on_sc=False),
        scratch_types=[
            pltpu.VMEM((PCHUNK,), jnp.int32),
            pltpu.VMEM((PCHUNK, H), jnp.float32),
            pltpu.VMEM((PCHUNK, H), jnp.float32),
            pltpu.VMEM_SHARED((GACC, H), jnp.float32),
            pltpu.VMEM_SHARED((GACC, H), jnp.float32),
        ],
    )
    def pool(h_hbm, b_hbm, zg_hbm, onesp_hbm, outp_hbm, outc_hbm,
             b_v, rows_v, ones_v, accp, accc):
        c = lax.axis_index("c")
        s = lax.axis_index("s")
        wid = s * NC + c
        pltpu.sync_copy(zg_hbm, accp.at[pl.ds(s * GRPT, GRPT)])
        pltpu.sync_copy(zg_hbm, accc.at[pl.ds(s * GRPT, GRPT)])
        pltpu.sync_copy(onesp_hbm, ones_v)
        plsc.subcore_barrier()
        nbase = wid * NPT

        def piter(i, carry):
            off = nbase + i * PCHUNK
            pltpu.sync_copy(b_hbm.at[pl.ds(off, PCHUNK)], b_v)
            pltpu.sync_copy(h_hbm.at[pl.ds(off, PCHUNK)], rows_v)
            pltpu.sync_copy(rows_v, accp.at[b_v], add=True)
            pltpu.sync_copy(ones_v, accc.at[b_v], add=True)
            return carry

        lax.fori_loop(0, NPT // PCHUNK, piter, 0)
        plsc.subcore_barrier()
        pltpu.sync_copy(accp.at[pl.ds(s * GRPT, GRPT)],
                        outp_hbm.at[c, pl.ds(s * GRPT, GRPT)])
        pltpu.sync_copy(accc.at[pl.ds(s * GRPT, GRPT)],
                        outc_hbm.at[c, pl.ds(s * GRPT, GRPT)])

    return pool


def _l1_body(x_ref, deg_ref, embp_ref, w1_ref, g_ref, dinv_ref):
    pid = pl.program_id(0)
    x = x_ref[...]
    tt = lax.broadcasted_iota(jnp.int32, (1, 16), 1).astype(jnp.float32)
    oh = (x == tt).astype(jnp.float32)
    table = jnp.dot(embp_ref[...], w1_ref[...],
                    preferred_element_type=jnp.float32,
                    precision=lax.Precision.HIGHEST)
    cnt = deg_ref[0, :, 0:1] + deg_ref[1, :, 0:1]
    rid = pid * ROW_BLK + lax.broadcasted_iota(jnp.int32, (ROW_BLK, 1), 0)
    valid = (rid % HALF) < REAL_HALF
    dinv = jnp.where(valid, lax.rsqrt(1.0 + cnt), 0.0)
    g = jnp.dot(oh, table, preferred_element_type=jnp.float32,
                precision=lax.Precision.HIGHEST) * dinv
    g_ref[0] = g[:, :32]
    g_ref[1] = g[:, 32:]
    dinv_ref[...] = dinv


@functools.lru_cache(maxsize=None)
def _build_l1():
    return pl.pallas_call(
        _l1_body,
        grid=(GRID,),
        in_specs=[
            pl.BlockSpec((ROW_BLK, 1), lambda i: (i, 0)),
            pl.BlockSpec((NC, ROW_BLK, DCOL), lambda i: (0, i, 0)),
            pl.BlockSpec((16, H), lambda i: (0, 0)),
            pl.BlockSpec((H, H), lambda i: (0, 0)),
        ],
        out_specs=[
            pl.BlockSpec((NC, ROW_BLK, 32), lambda i: (0, i, 0)),
            pl.BlockSpec((ROW_BLK, 1), lambda i: (i, 0)),
        ],
        out_shape=[
            jax.ShapeDtypeStruct((NC, NPAD, 32), jnp.float32),
            jax.ShapeDtypeStruct((NPAD, 1), jnp.float32),
        ],
    )


def _l23_body(s_ref, g_ref, dinv_ref, b_ref, w_ref, gout_ref):
    dinv = dinv_ref[...]
    sg = jnp.concatenate([s_ref[0] + g_ref[0], s_ref[1] + g_ref[1]], axis=1)
    h = jnp.maximum(dinv * sg + b_ref[...], 0.0)
    g = jnp.dot(h, w_ref[...], preferred_element_type=jnp.float32,
                precision=lax.Precision.HIGHEST) * dinv
    gout_ref[0] = g[:, :32]
    gout_ref[1] = g[:, 32:]


@functools.lru_cache(maxsize=None)
def _build_l23():
    return pl.pallas_call(
        _l23_body,
        grid=(GRID,),
        in_specs=[
            pl.BlockSpec((NC, ROW_BLK, 32), lambda i: (0, i, 0)),
            pl.BlockSpec((NC, ROW_BLK, 32), lambda i: (0, i, 0)),
            pl.BlockSpec((ROW_BLK, 1), lambda i: (i, 0)),
            pl.BlockSpec((1, H), lambda i: (0, 0)),
            pl.BlockSpec((H, H), lambda i: (0, 0)),
        ],
        out_specs=pl.BlockSpec((NC, ROW_BLK, 32), lambda i: (0, i, 0)),
        out_shape=jax.ShapeDtypeStruct((NC, NPAD, 32), jnp.float32),
    )


def _comb_body(s_ref, g_ref, dinv_ref, b_ref, h_ref):
    sg = jnp.concatenate([s_ref[0] + g_ref[0], s_ref[1] + g_ref[1]], axis=1)
    h_ref[...] = jnp.maximum(dinv_ref[...] * sg + b_ref[...], 0.0)


@functools.lru_cache(maxsize=None)
def _build_comb():
    return pl.pallas_call(
        _comb_body,
        grid=(GRID,),
        in_specs=[
            pl.BlockSpec((NC, ROW_BLK, 32), lambda i: (0, i, 0)),
            pl.BlockSpec((NC, ROW_BLK, 32), lambda i: (0, i, 0)),
            pl.BlockSpec((ROW_BLK, 1), lambda i: (i, 0)),
            pl.BlockSpec((1, H), lambda i: (0, 0)),
        ],
        out_specs=pl.BlockSpec((ROW_BLK, H), lambda i: (i, 0)),
        out_shape=jax.ShapeDtypeStruct((NPAD, H), jnp.float32),
    )


def _head_body(p_ref, cnt_ref, conc_ref, wc_ref, bc_ref,
               wf1_ref, bf1_ref, wf2_ref, bf2_ref, o_ref):
    sums = p_ref[0] + p_ref[1]
    cnts = cnt_ref[0, :, 0:1] + cnt_ref[1, :, 0:1]
    ge = sums[:B] / jnp.maximum(cnts[:B], 1.0)
    conc_e = conc_ref[...] * wc_ref[...] + bc_ref[...]
    h2 = jnp.maximum(
        jnp.dot(ge, wf1_ref[:H], preferred_element_type=jnp.float32,
                    precision=lax.Precision.HIGHEST)
        + jnp.dot(conc_e, wf1_ref[H:], preferred_element_type=jnp.float32,
                    precision=lax.Precision.HIGHEST)
        + bf1_ref[...], 0.0)
    o_ref[...] = jnp.dot(h2, wf2_ref[...],
                         preferred_element_type=jnp.float32,
                    precision=lax.Precision.HIGHEST) + bf2_ref[...]


@functools.lru_cache(maxsize=None)
def _build_head():
    return pl.pallas_call(
        _head_body,
        out_shape=jax.ShapeDtypeStruct((B, 128), jnp.float32),
    )


def kernel(x, edge_index, batch, concentration, emb,
           W1, b1, W2, b2, W3, b3, Wc, bc, Wf1, bf1, Wf2, bf2):
    f32 = jnp.float32
    src = edge_index[0]
    dst = edge_index[1]
    srcp = src + PAD * (src >= REAL_HALF).astype(jnp.int32)
    dstp = dst + PAD * (dst >= REAL_HALF).astype(jnp.int32)
    srcp = jnp.concatenate([srcp, jnp.zeros((EPAD - E,), jnp.int32)])
    dstp = jnp.concatenate([dstp, jnp.full((EPAD - E,), TRASH, jnp.int32)])

    padi = jnp.zeros((PAD,), jnp.int32)
    xp = jnp.concatenate([x[:REAL_HALF], padi, x[REAL_HALF:], padi])
    xp = xp.astype(f32).reshape(NPAD, 1)
    padb = jnp.full((PAD,), GTRASH, jnp.int32)
    batchp = jnp.concatenate([batch[:REAL_HALF], padb, batch[REAL_HALF:], padb])

    ztile32 = jnp.zeros((RPT, 32), f32)
    ztile16 = jnp.zeros((RPT, DCOL), f32)
    zg = jnp.zeros((GRPT, H), f32)
    ones16 = jnp.ones((CHUNK, DCOL), f32)
    ones_p = jnp.ones((PCHUNK, H), f32)
    embp = jnp.pad(emb, ((0, 16 - T), (0, 0)))
    wf2p = jnp.pad(Wf2, ((0, 0), (0, 128 - O)))
    bf2p = jnp.pad(bf2, (0, 128 - O)).reshape(1, 128)

    degr = _build_deg()(dstp, ztile16, ones16)
    g1, dinv = _build_l1()(xp, degr, embp, W1)
    s1 = _build_agg()(g1, srcp, dstp, ztile32)
    g2 = _build_l23()(s1, g1, dinv, b1.reshape(1, H), W2)
    s2 = _build_agg()(g2, srcp, dstp, ztile32)
    g3 = _build_l23()(s2, g2, dinv, b2.reshape(1, H), W3)
    s3 = _build_agg()(g3, srcp, dstp, ztile32)
    h3 = _build_comb()(s3, g3, dinv, b3.reshape(1, H))
    p, cnt = _build_pool()(h3, batchp, zg, ones_p)
    outp = _build_head()(p, cnt, concentration.reshape(B, 1), Wc,
                         bc.reshape(1, H), Wf1, bf1.reshape(1, H), wf2p, bf2p)
    return outp[:, :O]


# speedup vs baseline: 10.8730x; 1.3121x over previous
import functools

import jax
import jax.numpy as jnp
from jax import lax
from jax.experimental import pallas as pl
from jax.experimental.pallas import tpu as pltpu
from jax.experimental.pallas import tpu_sc as plsc

N = 50000
E = 800000
B = 512
T = 10
H = 64
O = 15

NC = 2
NS = 16
NW = NC * NS

REAL_HALF = 25000
HALF = 25088
NPAD = 2 * HALF
PAD = HALF - REAL_HALF
TRASH = 25040
RPT = NPAD // NS

EPAD = 819200
EPT = EPAD // NS
EPC = EPAD // NC
EPW = EPC // NS
CHUNK = 256
DCOL = 16

GACC = 640
GTRASH = 520
GRPT = GACC // NS
PCHUNK = 112
NPT = NPAD // NW

ROW_BLK = 512
GRID = NPAD // ROW_BLK


def _sc_mesh():
    return plsc.VectorSubcoreMesh(
        core_axis_name="c", subcore_axis_name="s", num_cores=NC, num_subcores=NS
    )


@functools.lru_cache(maxsize=None)
def _build_agg():
    @functools.partial(
        pl.kernel,
        out_type=jax.ShapeDtypeStruct((NC, NPAD, 32), jnp.float32),
        mesh=_sc_mesh(),
        compiler_params=pltpu.CompilerParams(use_tc_tiling_on_sc=False),
        scratch_types=[
            pltpu.VMEM((CHUNK,), jnp.int32),
            pltpu.VMEM((CHUNK,), jnp.int32),
            pltpu.VMEM((CHUNK, 32), jnp.float32),
            pltpu.VMEM_SHARED((NPAD, 32), jnp.float32),
            pltpu.SemaphoreType.DMA,
        ],
    )
    def agg(g_hbm, src_hbm, dst_hbm, ztile_hbm, out_hbm,
            src_v, d_v, rows_v, acc, sem):
        c = lax.axis_index("c")
        s = lax.axis_index("s")
        pltpu.sync_copy(ztile_hbm, acc.at[pl.ds(s * RPT, RPT)])
        plsc.subcore_barrier()
        ebase = s * EPT

        def eiter(i, carry):
            off = ebase + i * CHUNK
            pltpu.sync_copy(src_hbm.at[pl.ds(off, CHUNK)], src_v)
            pltpu.sync_copy(dst_hbm.at[pl.ds(off, CHUNK)], d_v)
            pltpu.async_copy(g_hbm.at[c].at[src_v], rows_v, sem).wait()
            pltpu.sync_copy(rows_v, acc.at[d_v], add=True)
            return carry

        lax.fori_loop(0, EPT // CHUNK, eiter, 0)
        plsc.subcore_barrier()
        pltpu.sync_copy(
            acc.at[pl.ds(s * RPT, RPT)],
            out_hbm.at[c, pl.ds(s * RPT, RPT)],
        )

    return agg


@functools.lru_cache(maxsize=None)
def _build_deg():
    @functools.partial(
        pl.kernel,
        out_type=jax.ShapeDtypeStruct((NC, NPAD, DCOL), jnp.float32),
        mesh=_sc_mesh(),
        compiler_params=pltpu.CompilerParams(use_tc_tiling_on_sc=False),
        scratch_types=[
            pltpu.VMEM((CHUNK,), jnp.int32),
            pltpu.VMEM((CHUNK, DCOL), jnp.float32),
            pltpu.VMEM_SHARED((NPAD, DCOL), jnp.float32),
        ],
    )
    def deg(dst_hbm, ztile_hbm, ones_hbm, out_hbm, d_v, ones_v, acc):
        c = lax.axis_index("c")
        s = lax.axis_index("s")
        pltpu.sync_copy(ztile_hbm, acc.at[pl.ds(s * RPT, RPT)])
        pltpu.sync_copy(ones_hbm, ones_v)
        plsc.subcore_barrier()
        ebase = c * EPC + s * EPW

        def eiter(i, carry):
            off = ebase + i * CHUNK
            pltpu.sync_copy(dst_hbm.at[pl.ds(off, CHUNK)], d_v)
            pltpu.sync_copy(ones_v, acc.at[d_v], add=True)
            return carry

        lax.fori_loop(0, EPW // CHUNK, eiter, 0)
        plsc.subcore_barrier()
        pltpu.sync_copy(
            acc.at[pl.ds(s * RPT, RPT)],
            out_hbm.at[c, pl.ds(s * RPT, RPT)],
        )

    return deg


@functools.lru_cache(maxsize=None)
def _build_pool():
    @functools.partial(
        pl.kernel,
        out_type=(
            jax.ShapeDtypeStruct((NC, GACC, H), jnp.float32),
            jax.ShapeDtypeStruct((NC, GACC, H), jnp.float32),
        ),
        mesh=_sc_mesh(),
        compiler_params=pltpu.CompilerParams(use_tc_tiling_on_sc=False),
        scratch_types=[
            pltpu.VMEM((PCHUNK,), jnp.int32),
            pltpu.VMEM((PCHUNK, H), jnp.float32),
            pltpu.VMEM((PCHUNK, H), jnp.float32),
            pltpu.VMEM_SHARED((GACC, H), jnp.float32),
            pltpu.VMEM_SHARED((GACC, H), jnp.float32),
        ],
    )
    def pool(h_hbm, b_hbm, zg_hbm, onesp_hbm, outp_hbm, outc_hbm,
             b_v, rows_v, ones_v, accp, accc):
        c = lax.axis_index("c")
        s = lax.axis_index("s")
        wid = s * NC + c
        pltpu.sync_copy(zg_hbm, accp.at[pl.ds(s * GRPT, GRPT)])
        pltpu.sync_copy(zg_hbm, accc.at[pl.ds(s * GRPT, GRPT)])
        pltpu.sync_copy(onesp_hbm, ones_v)
        plsc.subcore_barrier()
        nbase = wid * NPT

        def piter(i, carry):
            off = nbase + i * PCHUNK
            pltpu.sync_copy(b_hbm.at[pl.ds(off, PCHUNK)], b_v)
            pltpu.sync_copy(h_hbm.at[pl.ds(off, PCHUNK)], rows_v)
            pltpu.sync_copy(rows_v, accp.at[b_v], add=True)
            pltpu.sync_copy(ones_v, accc.at[b_v], add=True)
            return carry

        lax.fori_loop(0, NPT // PCHUNK, piter, 0)
        plsc.subcore_barrier()
        pltpu.sync_copy(accp.at[pl.ds(s * GRPT, GRPT)],
                        outp_hbm.at[c, pl.ds(s * GRPT, GRPT)])
        pltpu.sync_copy(accc.at[pl.ds(s * GRPT, GRPT)],
                        outc_hbm.at[c, pl.ds(s * GRPT, GRPT)])

    return pool


def _l1_body(x_ref, deg_ref, embp_ref, w1_ref, g_ref, dinv_ref):
    pid = pl.program_id(0)
    x = x_ref[...]
    tt = lax.broadcasted_iota(jnp.int32, (1, 16), 1).astype(jnp.float32)
    oh = (x == tt).astype(jnp.float32)
    table = jnp.dot(embp_ref[...], w1_ref[...],
                    preferred_element_type=jnp.float32,
                    precision=lax.Precision.HIGHEST)
    cnt = deg_ref[0, :, 0:1] + deg_ref[1, :, 0:1]
    rid = pid * ROW_BLK + lax.broadcasted_iota(jnp.int32, (ROW_BLK, 1), 0)
    valid = (rid % HALF) < REAL_HALF
    dinv = jnp.where(valid, lax.rsqrt(1.0 + cnt), 0.0)
    g = jnp.dot(oh, table, preferred_element_type=jnp.float32,
                precision=lax.Precision.HIGHEST) * dinv
    g_ref[0] = g[:, :32]
    g_ref[1] = g[:, 32:]
    dinv_ref[...] = dinv


@functools.lru_cache(maxsize=None)
def _build_l1():
    return pl.pallas_call(
        _l1_body,
        grid=(GRID,),
        in_specs=[
            pl.BlockSpec((ROW_BLK, 1), lambda i: (i, 0)),
            pl.BlockSpec((NC, ROW_BLK, DCOL), lambda i: (0, i, 0)),
            pl.BlockSpec((16, H), lambda i: (0, 0)),
            pl.BlockSpec((H, H), lambda i: (0, 0)),
        ],
        out_specs=[
            pl.BlockSpec((NC, ROW_BLK, 32), lambda i: (0, i, 0)),
            pl.BlockSpec((ROW_BLK, 1), lambda i: (i, 0)),
        ],
        out_shape=[
            jax.ShapeDtypeStruct((NC, NPAD, 32), jnp.float32),
            jax.ShapeDtypeStruct((NPAD, 1), jnp.float32),
        ],
    )


def _l23_body(s_ref, g_ref, dinv_ref, b_ref, w_ref, gout_ref):
    dinv = dinv_ref[...]
    sg = jnp.concatenate([s_ref[0] + g_ref[0], s_ref[1] + g_ref[1]], axis=1)
    h = jnp.maximum(dinv * sg + b_ref[...], 0.0)
    g = jnp.dot(h, w_ref[...], preferred_element_type=jnp.float32,
                precision=lax.Precision.HIGHEST) * dinv
    gout_ref[0] = g[:, :32]
    gout_ref[1] = g[:, 32:]


@functools.lru_cache(maxsize=None)
def _build_l23():
    return pl.pallas_call(
        _l23_body,
        grid=(GRID,),
        in_specs=[
            pl.BlockSpec((NC, ROW_BLK, 32), lambda i: (0, i, 0)),
            pl.BlockSpec((NC, ROW_BLK, 32), lambda i: (0, i, 0)),
            pl.BlockSpec((ROW_BLK, 1), lambda i: (i, 0)),
            pl.BlockSpec((1, H), lambda i: (0, 0)),
            pl.BlockSpec((H, H), lambda i: (0, 0)),
        ],
        out_specs=pl.BlockSpec((NC, ROW_BLK, 32), lambda i: (0, i, 0)),
        out_shape=jax.ShapeDtypeStruct((NC, NPAD, 32), jnp.float32),
    )


def _comb_body(s_ref, g_ref, dinv_ref, b_ref, h_ref):
    sg = jnp.concatenate([s_ref[0] + g_ref[0], s_ref[1] + g_ref[1]], axis=1)
    h_ref[...] = jnp.maximum(dinv_ref[...] * sg + b_ref[...], 0.0)


@functools.lru_cache(maxsize=None)
def _build_comb():
    return pl.pallas_call(
        _comb_body,
        grid=(GRID,),
        in_specs=[
            pl.BlockSpec((NC, ROW_BLK, 32), lambda i: (0, i, 0)),
            pl.BlockSpec((NC, ROW_BLK, 32), lambda i: (0, i, 0)),
            pl.BlockSpec((ROW_BLK, 1), lambda i: (i, 0)),
            pl.BlockSpec((1, H), lambda i: (0, 0)),
        ],
        out_specs=pl.BlockSpec((ROW_BLK, H), lambda i: (i, 0)),
        out_shape=jax.ShapeDtypeStruct((NPAD, H), jnp.float32),
    )


def _head_body(p_ref, cnt_ref, conc_ref, wc_ref, bc_ref,
               wf1_ref, bf1_ref, wf2_ref, bf2_ref, o_ref):
    sums = p_ref[0] + p_ref[1]
    cnts = cnt_ref[0, :, 0:1] + cnt_ref[1, :, 0:1]
    ge = sums[:B] / jnp.maximum(cnts[:B], 1.0)
    conc_e = conc_ref[...] * wc_ref[...] + bc_ref[...]
    h2 = jnp.maximum(
        jnp.dot(ge, wf1_ref[:H], preferred_element_type=jnp.float32,
                    precision=lax.Precision.HIGHEST)
        + jnp.dot(conc_e, wf1_ref[H:], preferred_element_type=jnp.float32,
                    precision=lax.Precision.HIGHEST)
        + bf1_ref[...], 0.0)
    o_ref[...] = jnp.dot(h2, wf2_ref[...],
                         preferred_element_type=jnp.float32,
                    precision=lax.Precision.HIGHEST) + bf2_ref[...]


@functools.lru_cache(maxsize=None)
def _build_head():
    return pl.pallas_call(
        _head_body,
        out_shape=jax.ShapeDtypeStruct((B, 128), jnp.float32),
    )


def kernel(x, edge_index, batch, concentration, emb,
           W1, b1, W2, b2, W3, b3, Wc, bc, Wf1, bf1, Wf2, bf2):
    f32 = jnp.float32
    src = edge_index[0]
    dst = edge_index[1]
    srcp = src + PAD * (src >= REAL_HALF).astype(jnp.int32)
    dstp = dst + PAD * (dst >= REAL_HALF).astype(jnp.int32)
    srcp = jnp.concatenate([srcp, jnp.zeros((EPAD - E,), jnp.int32)])
    dstp = jnp.concatenate([dstp, jnp.full((EPAD - E,), TRASH, jnp.int32)])

    padi = jnp.zeros((PAD,), jnp.int32)
    xp = jnp.concatenate([x[:REAL_HALF], padi, x[REAL_HALF:], padi])
    xp = xp.astype(f32).reshape(NPAD, 1)
    padb = jnp.full((PAD,), GTRASH, jnp.int32)
    batchp = jnp.concatenate([batch[:REAL_HALF], padb, batch[REAL_HALF:], padb])

    ztile32 = jnp.zeros((RPT, 32), f32)
    ztile16 = jnp.zeros((RPT, DCOL), f32)
    zg = jnp.zeros((GRPT, H), f32)
    ones16 = jnp.ones((CHUNK, DCOL), f32)
    ones_p = jnp.ones((PCHUNK, H), f32)
    embp = jnp.pad(emb, ((0, 16 - T), (0, 0)))
    wf2p = jnp.pad(Wf2, ((0, 0), (0, 128 - O)))
    bf2p = jnp.pad(bf2, (0, 128 - O)).reshape(1, 128)

    degr = _build_deg()(dstp, ztile16, ones16)
    g1, dinv = _build_l1()(xp, degr, embp, W1)
    s1 = _build_agg()(g1, srcp, dstp, ztile32)
    g2 = _build_l23()(s1, g1, dinv, b1.reshape(1, H), W2)
    s2 = _build_agg()(g2, srcp, dstp, ztile32)
    g3 = _build_l23()(s2, g2, dinv, b2.reshape(1, H), W3)
    s3 = _build_agg()(g3, srcp, dstp, ztile32)
    h3 = _build_comb()(s3, g3, dinv, b3.reshape(1, H))
    p, cnt = _build_pool()(h3, batchp, zg, ones_p)
    outp = _build_head()(p, cnt, concentration.reshape(B, 1), Wc,
                         bc.reshape(1, H), Wf1, bf1.reshape(1, H), wf2p, bf2p)
    return outp[:, :O]


# speedup vs baseline: 12.8955x; 1.1860x over previous
import functools

import jax
import jax.numpy as jnp
from jax import lax
from jax.experimental import pallas as pl
from jax.experimental.pallas import tpu as pltpu
from jax.experimental.pallas import tpu_sc as plsc

N = 50000
E = 800000
B = 512
T = 10
H = 64
O = 15

NC = 2
NS = 16
NW = NC * NS

REAL_HALF = 25000
HALF = 25088
NPAD = 2 * HALF
PAD = HALF - REAL_HALF
TRASH = 25040
RPT = NPAD // NS

EPAD = 819200
EPT = EPAD // NS
EPC = EPAD // NC
EPW = EPC // NS
CHUNK = 512
DCOL = 16

GACC = 640
GTRASH = 520
GRPT = GACC // NS
PCHUNK = 112
NPT = NPAD // NW

ROW_BLK = 512
GRID = NPAD // ROW_BLK


def _sc_mesh():
    return plsc.VectorSubcoreMesh(
        core_axis_name="c", subcore_axis_name="s", num_cores=NC, num_subcores=NS
    )


@functools.lru_cache(maxsize=None)
def _build_agg():
    @functools.partial(
        pl.kernel,
        out_type=jax.ShapeDtypeStruct((NC, NPAD, 32), jnp.float32),
        mesh=_sc_mesh(),
        compiler_params=pltpu.CompilerParams(use_tc_tiling_on_sc=False),
        scratch_types=[
            pltpu.VMEM((CHUNK,), jnp.int32),
            pltpu.VMEM((CHUNK,), jnp.int32),
            pltpu.VMEM((CHUNK, 32), jnp.float32),
            pltpu.VMEM_SHARED((NPAD, 32), jnp.float32),
            pltpu.SemaphoreType.DMA,
        ],
    )
    def agg(g_hbm, src_hbm, dst_hbm, ztile_hbm, out_hbm,
            src_v, d_v, rows_v, acc, sem):
        c = lax.axis_index("c")
        s = lax.axis_index("s")
        pltpu.sync_copy(ztile_hbm, acc.at[pl.ds(s * RPT, RPT)])
        plsc.subcore_barrier()
        ebase = s * EPT

        def eiter(i, carry):
            off = ebase + i * CHUNK
            pltpu.sync_copy(src_hbm.at[pl.ds(off, CHUNK)], src_v)
            pltpu.sync_copy(dst_hbm.at[pl.ds(off, CHUNK)], d_v)
            pltpu.async_copy(g_hbm.at[c].at[src_v], rows_v, sem).wait()
            pltpu.sync_copy(rows_v, acc.at[d_v], add=True)
            return carry

        lax.fori_loop(0, EPT // CHUNK, eiter, 0)
        plsc.subcore_barrier()
        pltpu.sync_copy(
            acc.at[pl.ds(s * RPT, RPT)],
            out_hbm.at[c, pl.ds(s * RPT, RPT)],
        )

    return agg


@functools.lru_cache(maxsize=None)
def _build_deg():
    @functools.partial(
        pl.kernel,
        out_type=jax.ShapeDtypeStruct((NC, NPAD, DCOL), jnp.float32),
        mesh=_sc_mesh(),
        compiler_params=pltpu.CompilerParams(use_tc_tiling_on_sc=False),
        scratch_types=[
            pltpu.VMEM((CHUNK,), jnp.int32),
            pltpu.VMEM((CHUNK, DCOL), jnp.float32),
            pltpu.VMEM_SHARED((NPAD, DCOL), jnp.float32),
        ],
    )
    def deg(dst_hbm, ztile_hbm, ones_hbm, out_hbm, d_v, ones_v, acc):
        c = lax.axis_index("c")
        s = lax.axis_index("s")
        pltpu.sync_copy(ztile_hbm, acc.at[pl.ds(s * RPT, RPT)])
        pltpu.sync_copy(ones_hbm, ones_v)
        plsc.subcore_barrier()
        ebase = c * EPC + s * EPW

        def eiter(i, carry):
            off = ebase + i * CHUNK
            pltpu.sync_copy(dst_hbm.at[pl.ds(off, CHUNK)], d_v)
            pltpu.sync_copy(ones_v, acc.at[d_v], add=True)
            return carry

        lax.fori_loop(0, EPW // CHUNK, eiter, 0)
        plsc.subcore_barrier()
        pltpu.sync_copy(
            acc.at[pl.ds(s * RPT, RPT)],
            out_hbm.at[c, pl.ds(s * RPT, RPT)],
        )

    return deg


@functools.lru_cache(maxsize=None)
def _build_pool():
    @functools.partial(
        pl.kernel,
        out_type=(
            jax.ShapeDtypeStruct((NC, GACC, H), jnp.float32),
            jax.ShapeDtypeStruct((NC, GACC, H), jnp.float32),
        ),
        mesh=_sc_mesh(),
        compiler_params=pltpu.CompilerParams(use_tc_tiling_on_sc=False),
        scratch_types=[
            pltpu.VMEM((PCHUNK,), jnp.int32),
            pltpu.VMEM((PCHUNK, H), jnp.float32),
            pltpu.VMEM((PCHUNK, H), jnp.float32),
            pltpu.VMEM_SHARED((GACC, H), jnp.float32),
            pltpu.VMEM_SHARED((GACC, H), jnp.float32),
        ],
    )
    def pool(h_hbm, b_hbm, zg_hbm, onesp_hbm, outp_hbm, outc_hbm,
             b_v, rows_v, ones_v, accp, accc):
        c = lax.axis_index("c")
        s = lax.axis_index("s")
        wid = s * NC + c
        pltpu.sync_copy(zg_hbm, accp.at[pl.ds(s * GRPT, GRPT)])
        pltpu.sync_copy(zg_hbm, accc.at[pl.ds(s * GRPT, GRPT)])
        pltpu.sync_copy(onesp_hbm, ones_v)
        plsc.subcore_barrier()
        nbase = wid * NPT

        def piter(i, carry):
            off = nbase + i * PCHUNK
            pltpu.sync_copy(b_hbm.at[pl.ds(off, PCHUNK)], b_v)
            pltpu.sync_copy(h_hbm.at[pl.ds(off, PCHUNK)], rows_v)
            pltpu.sync_copy(rows_v, accp.at[b_v], add=True)
            pltpu.sync_copy(ones_v, accc.at[b_v], add=True)
            return carry

        lax.fori_loop(0, NPT // PCHUNK, piter, 0)
        plsc.subcore_barrier()
        pltpu.sync_copy(accp.at[pl.ds(s * GRPT, GRPT)],
                        outp_hbm.at[c, pl.ds(s * GRPT, GRPT)])
        pltpu.sync_copy(accc.at[pl.ds(s * GRPT, GRPT)],
                        outc_hbm.at[c, pl.ds(s * GRPT, GRPT)])

    return pool


def _l1_body(x_ref, deg_ref, embp_ref, w1_ref, g_ref, dinv_ref):
    pid = pl.program_id(0)
    x = x_ref[...]
    tt = lax.broadcasted_iota(jnp.int32, (1, 16), 1).astype(jnp.float32)
    oh = (x == tt).astype(jnp.float32)
    table = jnp.dot(embp_ref[...], w1_ref[...],
                    preferred_element_type=jnp.float32,
                    precision=lax.Precision.HIGHEST)
    cnt = deg_ref[0, :, 0:1] + deg_ref[1, :, 0:1]
    rid = pid * ROW_BLK + lax.broadcasted_iota(jnp.int32, (ROW_BLK, 1), 0)
    valid = (rid % HALF) < REAL_HALF
    dinv = jnp.where(valid, lax.rsqrt(1.0 + cnt), 0.0)
    g = jnp.dot(oh, table, preferred_element_type=jnp.float32,
                precision=lax.Precision.HIGHEST) * dinv
    g_ref[0] = g[:, :32]
    g_ref[1] = g[:, 32:]
    dinv_ref[...] = dinv


@functools.lru_cache(maxsize=None)
def _build_l1():
    return pl.pallas_call(
        _l1_body,
        grid=(GRID,),
        in_specs=[
            pl.BlockSpec((ROW_BLK, 1), lambda i: (i, 0)),
            pl.BlockSpec((NC, ROW_BLK, DCOL), lambda i: (0, i, 0)),
            pl.BlockSpec((16, H), lambda i: (0, 0)),
            pl.BlockSpec((H, H), lambda i: (0, 0)),
        ],
        out_specs=[
            pl.BlockSpec((NC, ROW_BLK, 32), lambda i: (0, i, 0)),
            pl.BlockSpec((ROW_BLK, 1), lambda i: (i, 0)),
        ],
        out_shape=[
            jax.ShapeDtypeStruct((NC, NPAD, 32), jnp.float32),
            jax.ShapeDtypeStruct((NPAD, 1), jnp.float32),
        ],
    )


def _l23_body(s_ref, g_ref, dinv_ref, b_ref, w_ref, gout_ref):
    dinv = dinv_ref[...]
    sg = jnp.concatenate([s_ref[0] + g_ref[0], s_ref[1] + g_ref[1]], axis=1)
    h = jnp.maximum(dinv * sg + b_ref[...], 0.0)
    g = jnp.dot(h, w_ref[...], preferred_element_type=jnp.float32,
                precision=lax.Precision.HIGHEST) * dinv
    gout_ref[0] = g[:, :32]
    gout_ref[1] = g[:, 32:]


@functools.lru_cache(maxsize=None)
def _build_l23():
    return pl.pallas_call(
        _l23_body,
        grid=(GRID,),
        in_specs=[
            pl.BlockSpec((NC, ROW_BLK, 32), lambda i: (0, i, 0)),
            pl.BlockSpec((NC, ROW_BLK, 32), lambda i: (0, i, 0)),
            pl.BlockSpec((ROW_BLK, 1), lambda i: (i, 0)),
            pl.BlockSpec((1, H), lambda i: (0, 0)),
            pl.BlockSpec((H, H), lambda i: (0, 0)),
        ],
        out_specs=pl.BlockSpec((NC, ROW_BLK, 32), lambda i: (0, i, 0)),
        out_shape=jax.ShapeDtypeStruct((NC, NPAD, 32), jnp.float32),
    )


def _comb_body(s_ref, g_ref, dinv_ref, b_ref, h_ref):
    sg = jnp.concatenate([s_ref[0] + g_ref[0], s_ref[1] + g_ref[1]], axis=1)
    h_ref[...] = jnp.maximum(dinv_ref[...] * sg + b_ref[...], 0.0)


@functools.lru_cache(maxsize=None)
def _build_comb():
    return pl.pallas_call(
        _comb_body,
        grid=(GRID,),
        in_specs=[
            pl.BlockSpec((NC, ROW_BLK, 32), lambda i: (0, i, 0)),
            pl.BlockSpec((NC, ROW_BLK, 32), lambda i: (0, i, 0)),
            pl.BlockSpec((ROW_BLK, 1), lambda i: (i, 0)),
            pl.BlockSpec((1, H), lambda i: (0, 0)),
        ],
        out_specs=pl.BlockSpec((ROW_BLK, H), lambda i: (i, 0)),
        out_shape=jax.ShapeDtypeStruct((NPAD, H), jnp.float32),
    )


def _head_body(p_ref, cnt_ref, conc_ref, wc_ref, bc_ref,
               wf1_ref, bf1_ref, wf2_ref, bf2_ref, o_ref):
    sums = p_ref[0] + p_ref[1]
    cnts = cnt_ref[0, :, 0:1] + cnt_ref[1, :, 0:1]
    ge = sums[:B] / jnp.maximum(cnts[:B], 1.0)
    conc_e = conc_ref[...] * wc_ref[...] + bc_ref[...]
    h2 = jnp.maximum(
        jnp.dot(ge, wf1_ref[:H], preferred_element_type=jnp.float32,
                    precision=lax.Precision.HIGHEST)
        + jnp.dot(conc_e, wf1_ref[H:], preferred_element_type=jnp.float32,
                    precision=lax.Precision.HIGHEST)
        + bf1_ref[...], 0.0)
    o_ref[...] = jnp.dot(h2, wf2_ref[...],
                         preferred_element_type=jnp.float32,
                    precision=lax.Precision.HIGHEST) + bf2_ref[...]


@functools.lru_cache(maxsize=None)
def _build_head():
    return pl.pallas_call(
        _head_body,
        out_shape=jax.ShapeDtypeStruct((B, 128), jnp.float32),
    )


def kernel(x, edge_index, batch, concentration, emb,
           W1, b1, W2, b2, W3, b3, Wc, bc, Wf1, bf1, Wf2, bf2):
    f32 = jnp.float32
    src = edge_index[0]
    dst = edge_index[1]
    srcp = src + PAD * (src >= REAL_HALF).astype(jnp.int32)
    dstp = dst + PAD * (dst >= REAL_HALF).astype(jnp.int32)
    srcp = jnp.concatenate([srcp, jnp.zeros((EPAD - E,), jnp.int32)])
    dstp = jnp.concatenate([dstp, jnp.full((EPAD - E,), TRASH, jnp.int32)])

    padi = jnp.zeros((PAD,), jnp.int32)
    xp = jnp.concatenate([x[:REAL_HALF], padi, x[REAL_HALF:], padi])
    xp = xp.astype(f32).reshape(NPAD, 1)
    padb = jnp.full((PAD,), GTRASH, jnp.int32)
    batchp = jnp.concatenate([batch[:REAL_HALF], padb, batch[REAL_HALF:], padb])

    ztile32 = jnp.zeros((RPT, 32), f32)
    ztile16 = jnp.zeros((RPT, DCOL), f32)
    zg = jnp.zeros((GRPT, H), f32)
    ones16 = jnp.ones((CHUNK, DCOL), f32)
    ones_p = jnp.ones((PCHUNK, H), f32)
    embp = jnp.pad(emb, ((0, 16 - T), (0, 0)))
    wf2p = jnp.pad(Wf2, ((0, 0), (0, 128 - O)))
    bf2p = jnp.pad(bf2, (0, 128 - O)).reshape(1, 128)

    degr = _build_deg()(dstp, ztile16, ones16)
    g1, dinv = _build_l1()(xp, degr, embp, W1)
    s1 = _build_agg()(g1, srcp, dstp, ztile32)
    g2 = _build_l23()(s1, g1, dinv, b1.reshape(1, H), W2)
    s2 = _build_agg()(g2, srcp, dstp, ztile32)
    g3 = _build_l23()(s2, g2, dinv, b2.reshape(1, H), W3)
    s3 = _build_agg()(g3, srcp, dstp, ztile32)
    h3 = _build_comb()(s3, g3, dinv, b3.reshape(1, H))
    p, cnt = _build_pool()(h3, batchp, zg, ones_p)
    outp = _build_head()(p, cnt, concentration.reshape(B, 1), Wc,
                         bc.reshape(1, H), Wf1, bf1.reshape(1, H), wf2p, bf2p)
    return outp[:, :O]


# speedup vs baseline: 13.5648x; 1.0519x over previous
import functools

import jax
import jax.numpy as jnp
from jax import lax
from jax.experimental import pallas as pl
from jax.experimental.pallas import tpu as pltpu
from jax.experimental.pallas import tpu_sc as plsc

N = 50000
E = 800000
B = 512
T = 10
H = 64
O = 15

NC = 2
NS = 16
NW = NC * NS

REAL_HALF = 25000
HALF = 25088
NPAD = 2 * HALF
PAD = HALF - REAL_HALF
TRASH = 25040
RPT = NPAD // NS

EPAD = 819200
EPT = EPAD // NS
EPC = EPAD // NC
EPW = EPC // NS
CHUNK = 512
ACH = 256
DCOL = 16

GACC = 640
GTRASH = 520
GRPT = GACC // NS
PCHUNK = 112
NPT = NPAD // NW

ROW_BLK = 512
GRID = NPAD // ROW_BLK


def _sc_mesh():
    return plsc.VectorSubcoreMesh(
        core_axis_name="c", subcore_axis_name="s", num_cores=NC, num_subcores=NS
    )


@functools.lru_cache(maxsize=None)
def _build_agg():
    @functools.partial(
        pl.kernel,
        out_type=jax.ShapeDtypeStruct((NC, NPAD, 32), jnp.float32),
        mesh=_sc_mesh(),
        compiler_params=pltpu.CompilerParams(use_tc_tiling_on_sc=False),
        scratch_types=[
            pltpu.VMEM((ACH,), jnp.int32),
            pltpu.VMEM((ACH,), jnp.int32),
            pltpu.VMEM((ACH,), jnp.int32),
            pltpu.VMEM((ACH,), jnp.int32),
            pltpu.VMEM((ACH, 32), jnp.float32),
            pltpu.VMEM((ACH, 32), jnp.float32),
            pltpu.VMEM_SHARED((NPAD, 32), jnp.float32),
            pltpu.SemaphoreType.DMA,
            pltpu.SemaphoreType.DMA,
        ],
    )
    def agg(g_hbm, src_hbm, dst_hbm, ztile_hbm, out_hbm,
            sa_v, sb_v, da_v, db_v, rowsa_v, rowsb_v, acc, sema, semb):
        c = lax.axis_index("c")
        s = lax.axis_index("s")
        pltpu.sync_copy(ztile_hbm, acc.at[pl.ds(s * RPT, RPT)])
        plsc.subcore_barrier()
        ebase = s * EPT
        gc = g_hbm.at[c]

        def eiter(i, carry):
            off = ebase + i * (2 * ACH)
            pltpu.sync_copy(src_hbm.at[pl.ds(off, ACH)], sa_v)
            ha = pltpu.async_copy(gc.at[sa_v], rowsa_v, sema)
            pltpu.sync_copy(src_hbm.at[pl.ds(off + ACH, ACH)], sb_v)
            hb = pltpu.async_copy(gc.at[sb_v], rowsb_v, semb)
            pltpu.sync_copy(dst_hbm.at[pl.ds(off, ACH)], da_v)
            pltpu.sync_copy(dst_hbm.at[pl.ds(off + ACH, ACH)], db_v)
            ha.wait()
            pltpu.sync_copy(rowsa_v, acc.at[da_v], add=True)
            hb.wait()
            pltpu.sync_copy(rowsb_v, acc.at[db_v], add=True)
            return carry

        lax.fori_loop(0, EPT // (2 * ACH), eiter, 0)
        plsc.subcore_barrier()
        pltpu.sync_copy(
            acc.at[pl.ds(s * RPT, RPT)],
            out_hbm.at[c, pl.ds(s * RPT, RPT)],
        )

    return agg


@functools.lru_cache(maxsize=None)
def _build_deg():
    @functools.partial(
        pl.kernel,
        out_type=jax.ShapeDtypeStruct((NC, NPAD, DCOL), jnp.float32),
        mesh=_sc_mesh(),
        compiler_params=pltpu.CompilerParams(use_tc_tiling_on_sc=False),
        scratch_types=[
            pltpu.VMEM((CHUNK,), jnp.int32),
            pltpu.VMEM((CHUNK, DCOL), jnp.float32),
            pltpu.VMEM_SHARED((NPAD, DCOL), jnp.float32),
        ],
    )
    def deg(dst_hbm, ztile_hbm, ones_hbm, out_hbm, d_v, ones_v, acc):
        c = lax.axis_index("c")
        s = lax.axis_index("s")
        pltpu.sync_copy(ztile_hbm, acc.at[pl.ds(s * RPT, RPT)])
        pltpu.sync_copy(ones_hbm, ones_v)
        plsc.subcore_barrier()
        ebase = c * EPC + s * EPW

        def eiter(i, carry):
            off = ebase + i * CHUNK
            pltpu.sync_copy(dst_hbm.at[pl.ds(off, CHUNK)], d_v)
            pltpu.sync_copy(ones_v, acc.at[d_v], add=True)
            return carry

        lax.fori_loop(0, EPW // CHUNK, eiter, 0)
        plsc.subcore_barrier()
        pltpu.sync_copy(
            acc.at[pl.ds(s * RPT, RPT)],
            out_hbm.at[c, pl.ds(s * RPT, RPT)],
        )

    return deg


@functools.lru_cache(maxsize=None)
def _build_pool():
    @functools.partial(
        pl.kernel,
        out_type=(
            jax.ShapeDtypeStruct((NC, GACC, H), jnp.float32),
            jax.ShapeDtypeStruct((NC, GACC, H), jnp.float32),
        ),
        mesh=_sc_mesh(),
        compiler_params=pltpu.CompilerParams(use_tc_tiling_on_sc=False),
        scratch_types=[
            pltpu.VMEM((PCHUNK,), jnp.int32),
            pltpu.VMEM((PCHUNK, H), jnp.float32),
            pltpu.VMEM((PCHUNK, H), jnp.float32),
            pltpu.VMEM_SHARED((GACC, H), jnp.float32),
            pltpu.VMEM_SHARED((GACC, H), jnp.float32),
        ],
    )
    def pool(h_hbm, b_hbm, zg_hbm, onesp_hbm, outp_hbm, outc_hbm,
             b_v, rows_v, ones_v, accp, accc):
        c = lax.axis_index("c")
        s = lax.axis_index("s")
        wid = s * NC + c
        pltpu.sync_copy(zg_hbm, accp.at[pl.ds(s * GRPT, GRPT)])
        pltpu.sync_copy(zg_hbm, accc.at[pl.ds(s * GRPT, GRPT)])
        pltpu.sync_copy(onesp_hbm, ones_v)
        plsc.subcore_barrier()
        nbase = wid * NPT

        def piter(i, carry):
            off = nbase + i * PCHUNK
            pltpu.sync_copy(b_hbm.at[pl.ds(off, PCHUNK)], b_v)
            pltpu.sync_copy(h_hbm.at[pl.ds(off, PCHUNK)], rows_v)
            pltpu.sync_copy(rows_v, accp.at[b_v], add=True)
            pltpu.sync_copy(ones_v, accc.at[b_v], add=True)
            return carry

        lax.fori_loop(0, NPT // PCHUNK, piter, 0)
        plsc.subcore_barrier()
        pltpu.sync_copy(accp.at[pl.ds(s * GRPT, GRPT)],
                        outp_hbm.at[c, pl.ds(s * GRPT, GRPT)])
        pltpu.sync_copy(accc.at[pl.ds(s * GRPT, GRPT)],
                        outc_hbm.at[c, pl.ds(s * GRPT, GRPT)])

    return pool


def _l1_body(x_ref, deg_ref, embp_ref, w1_ref, g_ref, dinv_ref):
    pid = pl.program_id(0)
    x = x_ref[...]
    tt = lax.broadcasted_iota(jnp.int32, (1, 16), 1).astype(jnp.float32)
    oh = (x == tt).astype(jnp.float32)
    table = jnp.dot(embp_ref[...], w1_ref[...],
                    preferred_element_type=jnp.float32,
                    precision=lax.Precision.HIGHEST)
    cnt = deg_ref[0, :, 0:1] + deg_ref[1, :, 0:1]
    rid = pid * ROW_BLK + lax.broadcasted_iota(jnp.int32, (ROW_BLK, 1), 0)
    valid = (rid % HALF) < REAL_HALF
    dinv = jnp.where(valid, lax.rsqrt(1.0 + cnt), 0.0)
    g = jnp.dot(oh, table, preferred_element_type=jnp.float32,
                precision=lax.Precision.HIGHEST) * dinv
    g_ref[0] = g[:, :32]
    g_ref[1] = g[:, 32:]
    dinv_ref[...] = dinv


@functools.lru_cache(maxsize=None)
def _build_l1():
    return pl.pallas_call(
        _l1_body,
        grid=(GRID,),
        in_specs=[
            pl.BlockSpec((ROW_BLK, 1), lambda i: (i, 0)),
            pl.BlockSpec((NC, ROW_BLK, DCOL), lambda i: (0, i, 0)),
            pl.BlockSpec((16, H), lambda i: (0, 0)),
            pl.BlockSpec((H, H), lambda i: (0, 0)),
        ],
        out_specs=[
            pl.BlockSpec((NC, ROW_BLK, 32), lambda i: (0, i, 0)),
            pl.BlockSpec((ROW_BLK, 1), lambda i: (i, 0)),
        ],
        out_shape=[
            jax.ShapeDtypeStruct((NC, NPAD, 32), jnp.float32),
            jax.ShapeDtypeStruct((NPAD, 1), jnp.float32),
        ],
    )


def _l23_body(s_ref, g_ref, dinv_ref, b_ref, w_ref, gout_ref):
    dinv = dinv_ref[...]
    sg = jnp.concatenate([s_ref[0] + g_ref[0], s_ref[1] + g_ref[1]], axis=1)
    h = jnp.maximum(dinv * sg + b_ref[...], 0.0)
    g = jnp.dot(h, w_ref[...], preferred_element_type=jnp.float32,
                precision=lax.Precision.HIGHEST) * dinv
    gout_ref[0] = g[:, :32]
    gout_ref[1] = g[:, 32:]


@functools.lru_cache(maxsize=None)
def _build_l23():
    return pl.pallas_call(
        _l23_body,
        grid=(GRID,),
        in_specs=[
            pl.BlockSpec((NC, ROW_BLK, 32), lambda i: (0, i, 0)),
            pl.BlockSpec((NC, ROW_BLK, 32), lambda i: (0, i, 0)),
            pl.BlockSpec((ROW_BLK, 1), lambda i: (i, 0)),
            pl.BlockSpec((1, H), lambda i: (0, 0)),
            pl.BlockSpec((H, H), lambda i: (0, 0)),
        ],
        out_specs=pl.BlockSpec((NC, ROW_BLK, 32), lambda i: (0, i, 0)),
        out_shape=jax.ShapeDtypeStruct((NC, NPAD, 32), jnp.float32),
    )


def _comb_body(s_ref, g_ref, dinv_ref, b_ref, h_ref):
    sg = jnp.concatenate([s_ref[0] + g_ref[0], s_ref[1] + g_ref[1]], axis=1)
    h_ref[...] = jnp.maximum(dinv_ref[...] * sg + b_ref[...], 0.0)


@functools.lru_cache(maxsize=None)
def _build_comb():
    return pl.pallas_call(
        _comb_body,
        grid=(GRID,),
        in_specs=[
            pl.BlockSpec((NC, ROW_BLK, 32), lambda i: (0, i, 0)),
            pl.BlockSpec((NC, ROW_BLK, 32), lambda i: (0, i, 0)),
            pl.BlockSpec((ROW_BLK, 1), lambda i: (i, 0)),
            pl.BlockSpec((1, H), lambda i: (0, 0)),
        ],
        out_specs=pl.BlockSpec((ROW_BLK, H), lambda i: (i, 0)),
        out_shape=jax.ShapeDtypeStruct((NPAD, H), jnp.float32),
    )


def _head_body(p_ref, cnt_ref, conc_ref, wc_ref, bc_ref,
               wf1_ref, bf1_ref, wf2_ref, bf2_ref, o_ref):
    sums = p_ref[0] + p_ref[1]
    cnts = cnt_ref[0, :, 0:1] + cnt_ref[1, :, 0:1]
    ge = sums[:B] / jnp.maximum(cnts[:B], 1.0)
    conc_e = conc_ref[...] * wc_ref[...] + bc_ref[...]
    h2 = jnp.maximum(
        jnp.dot(ge, wf1_ref[:H], preferred_element_type=jnp.float32,
                    precision=lax.Precision.HIGHEST)
        + jnp.dot(conc_e, wf1_ref[H:], preferred_element_type=jnp.float32,
                    precision=lax.Precision.HIGHEST)
        + bf1_ref[...], 0.0)
    o_ref[...] = jnp.dot(h2, wf2_ref[...],
                         preferred_element_type=jnp.float32,
                    precision=lax.Precision.HIGHEST) + bf2_ref[...]


@functools.lru_cache(maxsize=None)
def _build_head():
    return pl.pallas_call(
        _head_body,
        out_shape=jax.ShapeDtypeStruct((B, 128), jnp.float32),
    )


def kernel(x, edge_index, batch, concentration, emb,
           W1, b1, W2, b2, W3, b3, Wc, bc, Wf1, bf1, Wf2, bf2):
    f32 = jnp.float32
    src = edge_index[0]
    dst = edge_index[1]
    srcp = src + PAD * (src >= REAL_HALF).astype(jnp.int32)
    dstp = dst + PAD * (dst >= REAL_HALF).astype(jnp.int32)
    srcp = jnp.concatenate([srcp, jnp.zeros((EPAD - E,), jnp.int32)])
    dstp = jnp.concatenate([dstp, jnp.full((EPAD - E,), TRASH, jnp.int32)])

    padi = jnp.zeros((PAD,), jnp.int32)
    xp = jnp.concatenate([x[:REAL_HALF], padi, x[REAL_HALF:], padi])
    xp = xp.astype(f32).reshape(NPAD, 1)
    padb = jnp.full((PAD,), GTRASH, jnp.int32)
    batchp = jnp.concatenate([batch[:REAL_HALF], padb, batch[REAL_HALF:], padb])

    ztile32 = jnp.zeros((RPT, 32), f32)
    ztile16 = jnp.zeros((RPT, DCOL), f32)
    zg = jnp.zeros((GRPT, H), f32)
    ones16 = jnp.ones((CHUNK, DCOL), f32)
    ones_p = jnp.ones((PCHUNK, H), f32)
    embp = jnp.pad(emb, ((0, 16 - T), (0, 0)))
    wf2p = jnp.pad(Wf2, ((0, 0), (0, 128 - O)))
    bf2p = jnp.pad(bf2, (0, 128 - O)).reshape(1, 128)

    degr = _build_deg()(dstp, ztile16, ones16)
    g1, dinv = _build_l1()(xp, degr, embp, W1)
    s1 = _build_agg()(g1, srcp, dstp, ztile32)
    g2 = _build_l23()(s1, g1, dinv, b1.reshape(1, H), W2)
    s2 = _build_agg()(g2, srcp, dstp, ztile32)
    g3 = _build_l23()(s2, g2, dinv, b2.reshape(1, H), W3)
    s3 = _build_agg()(g3, srcp, dstp, ztile32)
    h3 = _build_comb()(s3, g3, dinv, b3.reshape(1, H))
    p, cnt = _build_pool()(h3, batchp, zg, ones_p)
    outp = _build_head()(p, cnt, concentration.reshape(B, 1), Wc,
                         bc.reshape(1, H), Wf1, bf1.reshape(1, H), wf2p, bf2p)
    return outp[:, :O]
